# baseline (speedup 1.0000x reference)
import functools
import math

import jax
import jax.numpy as jnp
from jax import lax
from jax.experimental import pallas as pl
from jax.experimental.pallas import tpu as pltpu
from jax.experimental.pallas import tpu_sc as plsc

N = 10000
E = 160000
F = 128
B = 8
EPS = 1e-6

NC = 2
NS = 16
NW = NC * NS

E_PAD = 163840
EH_A = 65536
EH_B = E_PAD - EH_A
CHUNK = 128
SUP = 256
N_ACC = 10112
ROWS_PER_TILE = N_ACC // NS

RSQRT_F = 1.0 / math.sqrt(float(F))
DEG_NORM = 1.0 / math.sqrt(float(E) / float(N))
SQRT3 = math.sqrt(3.0)
SQRT15 = math.sqrt(15.0)
SQRT5_2 = math.sqrt(5.0) / 2.0
INV2SIG2 = 1.0 / (2.0 * 0.35 ** 2)


def _h_body(x_ref, w_ref, o_ref):
    o_ref[...] = jnp.dot(x_ref[...], w_ref[...],
                         preferred_element_type=jnp.float32) * RSQRT_F


def _input_linear(x0, W_in0):
    blk = 1000
    return pl.pallas_call(
        _h_body,
        grid=(N // blk,),
        in_specs=[pl.BlockSpec((blk, F), lambda i: (i, 0)),
                  pl.BlockSpec((F, F), lambda i: (0, 0))],
        out_specs=pl.BlockSpec((blk, F), lambda i: (i, 0)),
        out_shape=jax.ShapeDtypeStruct((N, F), jnp.float32),
    )(x0, W_in0)


def _make_gather_body(off, eh):
    nsup = eh // NW // SUP
    def body(h_hbm, idx2_hbm, out_hbm,
             idxg, ra, rb, rc,
             is0, gs0, gs1, gs2, ws0, ws1, ws2):
        c = lax.axis_index("c")
        s = lax.axis_index("s")
        wid = s * NC + c
        ebase = wid * (nsup * SUP)
        rbase = pl.multiple_of(off // CHUNK + wid * 2 * nsup, 8)
        rows = [ra, rb, rc]
        gsem = [gs0, gs1, gs2]
        wsem = [ws0, ws1, ws2]

        pltpu.sync_copy(idx2_hbm.at[pl.ds(rbase, 2 * nsup)], idxg)

        for j in range(nsup):
            b = j % 3
            if j >= 3:
                pltpu.make_async_copy(rows[b], out_hbm.at[pl.ds(0, SUP)],
                                      wsem[b]).wait()
            for q in range(2):
                pltpu.async_copy(h_hbm.at[idxg.at[2 * j + q]],
                                 rows[b].at[pl.ds(q * CHUNK, CHUNK)], gsem[b])
            for q in range(2):
                pltpu.make_async_copy(h_hbm.at[idxg.at[2 * j + q]],
                                      rows[b].at[pl.ds(q * CHUNK, CHUNK)],
                                      gsem[b]).wait()
            pltpu.async_copy(rows[b],
                             out_hbm.at[pl.ds(ebase + j * SUP, SUP)], wsem[b])

        for j in range(nsup - 3, nsup):
            b = j % 3
            pltpu.make_async_copy(rows[b], out_hbm.at[pl.ds(0, SUP)],
                                  wsem[b]).wait()
    return body


def _gather(h, src3, off, eh, half):
    mesh = plsc.VectorSubcoreMesh(core_axis_name="c", subcore_axis_name="s",
                                  num_cores=NC, num_subcores=NS)
    fn = functools.partial(
        pl.kernel,
        out_type=jax.ShapeDtypeStruct((eh, F), jnp.float32),
        mesh=mesh,
        scratch_types=(
            [pltpu.VMEM((2 * (eh // NW // SUP), CHUNK), jnp.int32)]
            + [pltpu.VMEM((SUP, F), jnp.float32)] * 3
            + [pltpu.SemaphoreType.DMA] * 7
        ),
        name=f"edge_gather_h{half}",
    )(_make_gather_body(off, eh))
    return fn(h, src3)


def _msg_body(hs_ref, ea_ref, w0_ref, w1_ref, w2_ref, o_ref):
    hs = hs_ref[...]
    ea = ea_ref[...]
    ex, ey, ez = ea[:, 0:1], ea[:, 1:2], ea[:, 2:3]
    d = jnp.sqrt(ex * ex + ey * ey + ez * ez + EPS)
    inv_d = 1.0 / d
    ux, uy, uz = ex * inv_d, ey * inv_d, ez * inv_d

    parts = []
    for b in range(B):
        cb = 2.5 * b / (B - 1)
        basis_b = jnp.exp(-((d - cb) ** 2) * INV2SIG2)
        parts.append(basis_b * hs)
    hb = jnp.concatenate(parts, axis=1).astype(jnp.bfloat16)

    s0 = jnp.dot(hb, w0_ref[...], preferred_element_type=jnp.float32) * RSQRT_F
    s1 = jnp.dot(hb, w1_ref[...], preferred_element_type=jnp.float32) * RSQRT_F
    s2 = jnp.dot(hb, w2_ref[...], preferred_element_type=jnp.float32) * RSQRT_F

    y1 = (SQRT3 * ux, SQRT3 * uy, SQRT3 * uz)
    y2 = (SQRT15 * ux * uy,
          SQRT15 * uy * uz,
          SQRT5_2 * (3.0 * uz * uz - 1.0),
          SQRT15 * ux * uz,
          (SQRT15 / 2.0) * (ux * ux - uy * uy))

    o_ref[0] = s0
    for m in range(3):
        o_ref[1 + m] = s1 * y1[m]
    for m in range(5):
        o_ref[4 + m] = s2 * y2[m]


def _edge_messages(hs, ea_full, off, Wr0f, Wr1f, Wr2f):
    blk = 512
    eh = hs.shape[0]
    ob = off // blk
    return pl.pallas_call(
        _msg_body,
        grid=(eh // blk,),
        in_specs=[pl.BlockSpec((blk, F), lambda i: (i, 0)),
                  pl.BlockSpec((blk, 3), lambda i: (i + ob, 0)),
                  pl.BlockSpec((B * F, F), lambda i: (0, 0)),
                  pl.BlockSpec((B * F, F), lambda i: (0, 0)),
                  pl.BlockSpec((B * F, F), lambda i: (0, 0))],
        out_specs=pl.BlockSpec((9, blk, F), lambda i: (0, i, 0)),
        out_shape=jax.ShapeDtypeStruct((9, eh, F), jnp.float32),
    )(hs, ea_full, Wr0f, Wr1f, Wr2f)


def _make_scatter_body(off, eh, half):
    nch = eh // NC // NS // CHUNK
    def body(msg_hbm, dst3_hbm, init_hbm, out_hbm,
             idxall, ma, mb, acc,
             fs0, fs1, ss0, ss1):
        c = lax.axis_index("c")
        s = lax.axis_index("s")
        msgs = [ma, mb]
        fsem = [fs0, fs1]
        ssem = [ss0, ss1]
        ebase = c * (eh // NC) + s * (nch * CHUNK)
        rbase = pl.multiple_of((off + ebase) // CHUNK, 8)

        pltpu.sync_copy(dst3_hbm.at[pl.ds(rbase, nch)], idxall)

        def pass_body(p, carry):
            if half == 0:
                pltpu.sync_copy(init_hbm,
                                acc.at[pl.ds(s * ROWS_PER_TILE,
                                             ROWS_PER_TILE)])
            else:
                pltpu.sync_copy(init_hbm.at[p, c,
                                            pl.ds(s * ROWS_PER_TILE,
                                                  ROWS_PER_TILE)],
                                acc.at[pl.ds(s * ROWS_PER_TILE,
                                             ROWS_PER_TILE)])
            plsc.subcore_barrier()

            def fetch(j, b):
                pltpu.async_copy(msg_hbm.at[p, pl.ds(ebase + j * CHUNK,
                                                     CHUNK)],
                                 msgs[b], fsem[b])

            def wait_fetch(b):
                pltpu.make_async_copy(msg_hbm.at[0, pl.ds(0, CHUNK)],
                                      msgs[b], fsem[b]).wait()

            def scat(j, b):
                pltpu.async_copy(msgs[b], acc.at[idxall.at[j]], ssem[b],
                                 add=True)

            def wait_scat(j, b):
                pltpu.make_async_copy(msgs[b], acc.at[idxall.at[j]],
                                      ssem[b]).wait()

            fetch(0, 0)
            for j in range(nch):
                b = j & 1
                wait_fetch(b)
                scat(j, b)
                if j + 1 < nch:
                    b1 = 1 - b
                    if j >= 1:
                        wait_scat(j - 1, b1)
                    fetch(j + 1, b1)
            wait_scat(nch - 2, 0)
            wait_scat(nch - 1, 1)
            plsc.subcore_barrier()

            for k in range(5):
                b = k & 1
                nr = CHUNK if k < 4 else (ROWS_PER_TILE - 4 * CHUNK)
                if k >= 2:
                    pltpu.make_async_copy(msgs[b].at[pl.ds(0, CHUNK)],
                                          out_hbm.at[p, c, pl.ds(0, CHUNK)],
                                          ssem[b]).wait()
                r0 = s * ROWS_PER_TILE + k * CHUNK
                pltpu.sync_copy(acc.at[pl.ds(r0, nr)],
                                msgs[b].at[pl.ds(0, nr)])
                pltpu.async_copy(msgs[b].at[pl.ds(0, nr)],
                                 out_hbm.at[p, c, pl.ds(r0, nr)], ssem[b])
            for k in range(3, 5):
                b = k & 1
                nr = CHUNK if k < 4 else (ROWS_PER_TILE - 4 * CHUNK)
                pltpu.make_async_copy(msgs[b].at[pl.ds(0, nr)],
                                      out_hbm.at[p, c, pl.ds(0, nr)],
                                      ssem[b]).wait()
            plsc.subcore_barrier()
            return carry

        lax.fori_loop(0, 9, pass_body, 0)
    return body


def _scatter(msg, dst3, init_arr, off, eh, half):
    mesh = plsc.VectorSubcoreMesh(core_axis_name="c", subcore_axis_name="s",
                                  num_cores=NC, num_subcores=NS)
    fn = functools.partial(
        pl.kernel,
        out_type=jax.ShapeDtypeStruct((9, NC, N_ACC, F), jnp.float32),
        mesh=mesh,
        scratch_types=(
            [pltpu.VMEM((eh // NC // NS // CHUNK, CHUNK), jnp.int32)]
            + [pltpu.VMEM((CHUNK, F), jnp.float32)] * 2
            + [pltpu.VMEM_SHARED((N_ACC, F), jnp.float32)]
            + [pltpu.SemaphoreType.DMA] * 4
        ),
        name=f"edge_scatter_h{half}",
    )(_make_scatter_body(off, eh, half))
    return fn(msg, dst3, init_arr)


def _out_body(pa_ref, w0_ref, w1_ref, w2_ref, o0_ref, o1_ref, o2_ref):
    pa = pa_ref[...]
    g = (pa[:, 0] + pa[:, 1]) * DEG_NORM

    a0 = g[0]
    a1 = [g[1 + m] for m in range(3)]
    a2 = [g[4 + m] for m in range(5)]

    rms0 = jnp.sqrt(jnp.mean(a0 * a0, axis=-1, keepdims=True) + EPS)
    n0 = a0 / rms0
    ss1 = sum(jnp.sum(t * t, axis=-1, keepdims=True) for t in a1)
    rms1 = jnp.sqrt(ss1 / (3.0 * F) + EPS)
    ss2 = sum(jnp.sum(t * t, axis=-1, keepdims=True) for t in a2)
    rms2 = jnp.sqrt(ss2 / (5.0 * F) + EPS)

    o0 = jnp.dot(n0, w0_ref[...], preferred_element_type=jnp.float32) * RSQRT_F
    o0_ref[...] = jax.nn.relu(o0)

    t1 = [jnp.dot(t / rms1, w1_ref[...], preferred_element_type=jnp.float32)
          * RSQRT_F for t in a1]
    nn1 = jnp.sqrt(sum(t * t for t in t1) + EPS)
    f1 = nn1 / (nn1 + EPS)
    o1_ref[...] = jnp.stack([t * f1 for t in t1],
                            axis=-1).reshape(t1[0].shape[0], 3 * F)

    t2 = [jnp.dot(t / rms2, w2_ref[...], preferred_element_type=jnp.float32)
          * RSQRT_F for t in a2]
    nn2 = jnp.sqrt(sum(t * t for t in t2) + EPS)
    f2 = nn2 / (nn2 + EPS)
    o2_ref[...] = jnp.stack([t * f2 for t in t2],
                            axis=-1).reshape(t2[0].shape[0], 5 * F)


def _node_epilogue(part, W_out0, W_out1, W_out2):
    blk = 200
    return pl.pallas_call(
        _out_body,
        grid=(N // blk,),
        in_specs=[pl.BlockSpec((9, NC, blk, F), lambda i: (0, 0, i, 0)),
                  pl.BlockSpec((F, F), lambda i: (0, 0)),
                  pl.BlockSpec((F, F), lambda i: (0, 0)),
                  pl.BlockSpec((F, F), lambda i: (0, 0))],
        out_specs=[pl.BlockSpec((blk, F), lambda i: (i, 0)),
                   pl.BlockSpec((blk, 3 * F), lambda i: (i, 0)),
                   pl.BlockSpec((blk, 5 * F), lambda i: (i, 0))],
        out_shape=[jax.ShapeDtypeStruct((N, F), jnp.float32),
                   jax.ShapeDtypeStruct((N, 3 * F), jnp.float32),
                   jax.ShapeDtypeStruct((N, 5 * F), jnp.float32)],
    )(part, W_out0, W_out1, W_out2)


def kernel(x, edge_index, edge_attr, W_in0, W_r0, W_r1, W_r2,
           W_out0, W_out1, W_out2):
    x0 = x[0]
    src = edge_index[0]
    dst = edge_index[1]
    pad = E_PAD - E
    src_p = jnp.concatenate([src, jnp.zeros((pad,), jnp.int32)])
    dst_p = jnp.concatenate([dst, jnp.full((pad,), N, jnp.int32)])
    ea_p = jnp.concatenate([edge_attr, jnp.zeros((pad, 3), jnp.float32)])
    src2 = src_p.reshape(E_PAD // CHUNK, CHUNK)
    dst2 = dst_p.reshape(E_PAD // CHUNK, CHUNK)
    zeros = jnp.zeros((ROWS_PER_TILE, F), jnp.float32)

    Wr0f = W_r0.transpose(0, 2, 1).reshape(B * F, F).astype(jnp.bfloat16)
    Wr1f = W_r1.transpose(0, 2, 1).reshape(B * F, F).astype(jnp.bfloat16)
    Wr2f = W_r2.transpose(0, 2, 1).reshape(B * F, F).astype(jnp.bfloat16)

    h = _input_linear(x0, W_in0)
    hs_a = _gather(h, src2, 0, EH_A, 0)
    msg_a = _edge_messages(hs_a, ea_p, 0, Wr0f, Wr1f, Wr2f)
    hs_b = _gather(h, src2, EH_A, EH_B, 1)
    msg_b = _edge_messages(hs_b, ea_p, EH_A, Wr0f, Wr1f, Wr2f)
    part_a = _scatter(msg_a, dst2, zeros, 0, EH_A, 0)
    part_b = _scatter(msg_b, dst2, part_a, EH_A, EH_B, 1)
    o0, o1, o2 = _node_epilogue(part_b, W_out0, W_out1, W_out2)
    return (o0, o1, o2)


# speedup vs baseline: 2.1272x; 2.1272x over previous
import functools
import math

import jax
import jax.numpy as jnp
from jax import lax
from jax.experimental import pallas as pl
from jax.experimental.pallas import tpu as pltpu
from jax.experimental.pallas import tpu_sc as plsc

N = 10000
E = 160000
F = 128
B = 8
EPS = 1e-6

NC = 2
NS = 16
NW = NC * NS

E_PAD = 163840
EH_A = 65536
EH_B = E_PAD - EH_A
CHUNK = 128
SUP = 256
N_ACC = 10112
ROWS_PER_TILE = N_ACC // NS

RSQRT_F = 1.0 / math.sqrt(float(F))
DEG_NORM = 1.0 / math.sqrt(float(E) / float(N))
SQRT3 = math.sqrt(3.0)
SQRT15 = math.sqrt(15.0)
SQRT5_2 = math.sqrt(5.0) / 2.0
INV2SIG2 = 1.0 / (2.0 * 0.35 ** 2)


def _h_body(x_ref, w_ref, o_ref):
    o_ref[...] = jnp.dot(x_ref[...], w_ref[...],
                         preferred_element_type=jnp.float32) * RSQRT_F


def _input_linear(x0, W_in0):
    blk = 1000
    return pl.pallas_call(
        _h_body,
        grid=(N // blk,),
        in_specs=[pl.BlockSpec((blk, F), lambda i: (i, 0)),
                  pl.BlockSpec((F, F), lambda i: (0, 0))],
        out_specs=pl.BlockSpec((blk, F), lambda i: (i, 0)),
        out_shape=jax.ShapeDtypeStruct((N, F), jnp.float32),
    )(x0, W_in0)


def _make_gather_body(off, eh):
    nsup = eh // NW // SUP
    def body(h_hbm, idx2_hbm, out_hbm,
             idxg, ra, rb, rc,
             is0, gs0, gs1, gs2, ws0, ws1, ws2):
        c = lax.axis_index("c")
        s = lax.axis_index("s")
        wid = s * NC + c
        ebase = wid * (nsup * SUP)
        rbase = pl.multiple_of(off // CHUNK + wid * 2 * nsup, 8)
        rows = [ra, rb, rc]
        gsem = [gs0, gs1, gs2]
        wsem = [ws0, ws1, ws2]

        pltpu.sync_copy(idx2_hbm.at[pl.ds(rbase, 2 * nsup)], idxg)

        for j in range(nsup):
            b = j % 3
            if j >= 3:
                pltpu.make_async_copy(rows[b], out_hbm.at[pl.ds(0, SUP)],
                                      wsem[b]).wait()
            for q in range(2):
                pltpu.async_copy(h_hbm.at[idxg.at[2 * j + q]],
                                 rows[b].at[pl.ds(q * CHUNK, CHUNK)], gsem[b])
            for q in range(2):
                pltpu.make_async_copy(h_hbm.at[idxg.at[2 * j + q]],
                                      rows[b].at[pl.ds(q * CHUNK, CHUNK)],
                                      gsem[b]).wait()
            pltpu.async_copy(rows[b],
                             out_hbm.at[pl.ds(ebase + j * SUP, SUP)], wsem[b])

        for j in range(nsup - 3, nsup):
            b = j % 3
            pltpu.make_async_copy(rows[b], out_hbm.at[pl.ds(0, SUP)],
                                  wsem[b]).wait()
    return body


def _gather(h, src3, off, eh, half):
    mesh = plsc.VectorSubcoreMesh(core_axis_name="c", subcore_axis_name="s",
                                  num_cores=NC, num_subcores=NS)
    fn = functools.partial(
        pl.kernel,
        out_type=jax.ShapeDtypeStruct((eh, F), jnp.float32),
        mesh=mesh,
        scratch_types=(
            [pltpu.VMEM((2 * (eh // NW // SUP), CHUNK), jnp.int32)]
            + [pltpu.VMEM((SUP, F), jnp.float32)] * 3
            + [pltpu.SemaphoreType.DMA] * 7
        ),
        name=f"edge_gather_h{half}",
    )(_make_gather_body(off, eh))
    return fn(h, src3)


def _msg_body(hs_ref, ea_ref, w0_ref, w1_ref, w2_ref, o_ref):
    hs = hs_ref[...]
    ea = ea_ref[...]
    ex, ey, ez = ea[:, 0:1], ea[:, 1:2], ea[:, 2:3]
    d = jnp.sqrt(ex * ex + ey * ey + ez * ez + EPS)
    inv_d = 1.0 / d
    ux, uy, uz = ex * inv_d, ey * inv_d, ez * inv_d

    parts = []
    for b in range(B):
        cb = 2.5 * b / (B - 1)
        basis_b = jnp.exp(-((d - cb) ** 2) * INV2SIG2)
        parts.append(basis_b * hs)
    hb = jnp.concatenate(parts, axis=1).astype(jnp.bfloat16)

    s0 = jnp.dot(hb, w0_ref[...], preferred_element_type=jnp.float32) * RSQRT_F
    s1 = jnp.dot(hb, w1_ref[...], preferred_element_type=jnp.float32) * RSQRT_F
    s2 = jnp.dot(hb, w2_ref[...], preferred_element_type=jnp.float32) * RSQRT_F

    y1 = (SQRT3 * ux, SQRT3 * uy, SQRT3 * uz)
    y2 = (SQRT15 * ux * uy,
          SQRT15 * uy * uz,
          SQRT5_2 * (3.0 * uz * uz - 1.0),
          SQRT15 * ux * uz,
          (SQRT15 / 2.0) * (ux * ux - uy * uy))

    o_ref[0] = s0
    for m in range(3):
        o_ref[1 + m] = s1 * y1[m]
    for m in range(5):
        o_ref[4 + m] = s2 * y2[m]


def _edge_messages(hs, ea_full, off, Wr0f, Wr1f, Wr2f):
    blk = 512
    eh = hs.shape[0]
    ob = off // blk
    return pl.pallas_call(
        _msg_body,
        grid=(eh // blk,),
        in_specs=[pl.BlockSpec((blk, F), lambda i: (i, 0)),
                  pl.BlockSpec((blk, 3), lambda i: (i + ob, 0)),
                  pl.BlockSpec((B * F, F), lambda i: (0, 0)),
                  pl.BlockSpec((B * F, F), lambda i: (0, 0)),
                  pl.BlockSpec((B * F, F), lambda i: (0, 0))],
        out_specs=pl.BlockSpec((9, blk, F), lambda i: (0, i, 0)),
        out_shape=jax.ShapeDtypeStruct((9, eh, F), jnp.float32),
    )(hs, ea_full, Wr0f, Wr1f, Wr2f)


def _make_scatter_body(off, eh, half):
    nch = eh // NC // NS // CHUNK
    def body(msg_hbm, dst3_hbm, init_hbm, out_hbm,
             idxall, ma, mb, acc,
             fs0, fs1, ss0, ss1):
        c = lax.axis_index("c")
        s = lax.axis_index("s")
        msgs = [ma, mb]
        fsem = [fs0, fs1]
        ssem = [ss0, ss1]
        ebase = c * (eh // NC) + s * (nch * CHUNK)
        rbase = pl.multiple_of((off + ebase) // CHUNK, 8)

        pltpu.sync_copy(dst3_hbm.at[pl.ds(rbase, nch)], idxall)

        def pass_body(p, carry):
            if half == 0:
                pltpu.sync_copy(init_hbm,
                                acc.at[pl.ds(s * ROWS_PER_TILE,
                                             ROWS_PER_TILE)])
            else:
                pltpu.sync_copy(init_hbm.at[p, c,
                                            pl.ds(s * ROWS_PER_TILE,
                                                  ROWS_PER_TILE)],
                                acc.at[pl.ds(s * ROWS_PER_TILE,
                                             ROWS_PER_TILE)])
            plsc.subcore_barrier()

            def fetch(j, b):
                pltpu.async_copy(msg_hbm.at[p, pl.ds(ebase + j * CHUNK,
                                                     CHUNK)],
                                 msgs[b], fsem[b])

            def wait_fetch(b):
                pltpu.make_async_copy(msg_hbm.at[0, pl.ds(0, CHUNK)],
                                      msgs[b], fsem[b]).wait()

            def scat(j, b):
                pltpu.async_copy(msgs[b], acc.at[idxall.at[j]], ssem[b],
                                 add=True)

            def wait_scat(j, b):
                pltpu.make_async_copy(msgs[b], acc.at[idxall.at[j]],
                                      ssem[b]).wait()

            fetch(0, 0)
            for j in range(nch):
                b = j & 1
                wait_fetch(b)
                scat(j, b)
                if j + 1 < nch:
                    b1 = 1 - b
                    if j >= 1:
                        wait_scat(j - 1, b1)
                    fetch(j + 1, b1)
            wait_scat(nch - 2, 0)
            wait_scat(nch - 1, 1)
            plsc.subcore_barrier()

            for k in range(5):
                b = k & 1
                nr = CHUNK if k < 4 else (ROWS_PER_TILE - 4 * CHUNK)
                if k >= 2:
                    pltpu.make_async_copy(msgs[b].at[pl.ds(0, CHUNK)],
                                          out_hbm.at[p, c, pl.ds(0, CHUNK)],
                                          ssem[b]).wait()
                r0 = s * ROWS_PER_TILE + k * CHUNK
                pltpu.sync_copy(acc.at[pl.ds(r0, nr)],
                                msgs[b].at[pl.ds(0, nr)])
                pltpu.async_copy(msgs[b].at[pl.ds(0, nr)],
                                 out_hbm.at[p, c, pl.ds(r0, nr)], ssem[b])
            for k in range(3, 5):
                b = k & 1
                nr = CHUNK if k < 4 else (ROWS_PER_TILE - 4 * CHUNK)
                pltpu.make_async_copy(msgs[b].at[pl.ds(0, nr)],
                                      out_hbm.at[p, c, pl.ds(0, nr)],
                                      ssem[b]).wait()
            plsc.subcore_barrier()
            return carry

        lax.fori_loop(0, 9, pass_body, 0)
    return body


def _scatter(msg, dst3, init_arr, off, eh, half):
    mesh = plsc.VectorSubcoreMesh(core_axis_name="c", subcore_axis_name="s",
                                  num_cores=NC, num_subcores=NS)
    fn = functools.partial(
        pl.kernel,
        out_type=jax.ShapeDtypeStruct((9, NC, N_ACC, F), jnp.float32),
        mesh=mesh,
        scratch_types=(
            [pltpu.VMEM((eh // NC // NS // CHUNK, CHUNK), jnp.int32)]
            + [pltpu.VMEM((CHUNK, F), jnp.float32)] * 2
            + [pltpu.VMEM_SHARED((N_ACC, F), jnp.float32)]
            + [pltpu.SemaphoreType.DMA] * 4
        ),
        name=f"edge_scatter_h{half}",
    )(_make_scatter_body(off, eh, half))
    return fn(msg, dst3, init_arr)


def _out_body(pa_ref, w0_ref, w1_ref, w2_ref, o0_ref, o1_ref, o2_ref):
    pa = pa_ref[...]
    g = (pa[:, 0] + pa[:, 1]) * DEG_NORM

    a0 = g[0]
    a1 = [g[1 + m] for m in range(3)]
    a2 = [g[4 + m] for m in range(5)]

    rms0 = jnp.sqrt(jnp.mean(a0 * a0, axis=-1, keepdims=True) + EPS)
    n0 = a0 / rms0
    ss1 = sum(jnp.sum(t * t, axis=-1, keepdims=True) for t in a1)
    rms1 = jnp.sqrt(ss1 / (3.0 * F) + EPS)
    ss2 = sum(jnp.sum(t * t, axis=-1, keepdims=True) for t in a2)
    rms2 = jnp.sqrt(ss2 / (5.0 * F) + EPS)

    o0 = jnp.dot(n0, w0_ref[...], preferred_element_type=jnp.float32) * RSQRT_F
    o0_ref[...] = jax.nn.relu(o0)

    t1 = [jnp.dot(t / rms1, w1_ref[...], preferred_element_type=jnp.float32)
          * RSQRT_F for t in a1]
    nn1 = jnp.sqrt(sum(t * t for t in t1) + EPS)
    f1 = nn1 / (nn1 + EPS)
    o1_ref[...] = jnp.concatenate([t * f1 for t in t1], axis=1)

    t2 = [jnp.dot(t / rms2, w2_ref[...], preferred_element_type=jnp.float32)
          * RSQRT_F for t in a2]
    nn2 = jnp.sqrt(sum(t * t for t in t2) + EPS)
    f2 = nn2 / (nn2 + EPS)
    o2_ref[...] = jnp.concatenate([t * f2 for t in t2], axis=1)


def _node_epilogue(part, W_out0, W_out1, W_out2):
    blk = 200
    return pl.pallas_call(
        _out_body,
        grid=(N // blk,),
        in_specs=[pl.BlockSpec((9, NC, blk, F), lambda i: (0, 0, i, 0)),
                  pl.BlockSpec((F, F), lambda i: (0, 0)),
                  pl.BlockSpec((F, F), lambda i: (0, 0)),
                  pl.BlockSpec((F, F), lambda i: (0, 0))],
        out_specs=[pl.BlockSpec((blk, F), lambda i: (i, 0)),
                   pl.BlockSpec((blk, 3 * F), lambda i: (i, 0)),
                   pl.BlockSpec((blk, 5 * F), lambda i: (i, 0))],
        out_shape=[jax.ShapeDtypeStruct((N, F), jnp.float32),
                   jax.ShapeDtypeStruct((N, 3 * F), jnp.float32),
                   jax.ShapeDtypeStruct((N, 5 * F), jnp.float32)],
    )(part, W_out0, W_out1, W_out2)


def kernel(x, edge_index, edge_attr, W_in0, W_r0, W_r1, W_r2,
           W_out0, W_out1, W_out2):
    x0 = x[0]
    src = edge_index[0]
    dst = edge_index[1]
    pad = E_PAD - E
    src_p = jnp.concatenate([src, jnp.zeros((pad,), jnp.int32)])
    dst_p = jnp.concatenate([dst, jnp.full((pad,), N, jnp.int32)])
    ea_p = jnp.concatenate([edge_attr, jnp.zeros((pad, 3), jnp.float32)])
    src2 = src_p.reshape(E_PAD // CHUNK, CHUNK)
    dst2 = dst_p.reshape(E_PAD // CHUNK, CHUNK)
    zeros = jnp.zeros((ROWS_PER_TILE, F), jnp.float32)

    Wr0f = W_r0.transpose(0, 2, 1).reshape(B * F, F).astype(jnp.bfloat16)
    Wr1f = W_r1.transpose(0, 2, 1).reshape(B * F, F).astype(jnp.bfloat16)
    Wr2f = W_r2.transpose(0, 2, 1).reshape(B * F, F).astype(jnp.bfloat16)

    h = _input_linear(x0, W_in0)
    hs_a = _gather(h, src2, 0, EH_A, 0)
    msg_a = _edge_messages(hs_a, ea_p, 0, Wr0f, Wr1f, Wr2f)
    hs_b = _gather(h, src2, EH_A, EH_B, 1)
    msg_b = _edge_messages(hs_b, ea_p, EH_A, Wr0f, Wr1f, Wr2f)
    part_a = _scatter(msg_a, dst2, zeros, 0, EH_A, 0)
    part_b = _scatter(msg_b, dst2, part_a, EH_A, EH_B, 1)
    o0, o1, o2 = _node_epilogue(part_b, W_out0, W_out1, W_out2)
    out1 = o1.reshape(N, 3, F).transpose(0, 2, 1).reshape(N, 3 * F)
    out2 = o2.reshape(N, 5, F).transpose(0, 2, 1).reshape(N, 5 * F)
    return (o0, out1, out2)


# speedup vs baseline: 2.1762x; 1.0230x over previous
import functools
import math

import jax
import jax.numpy as jnp
from jax import lax
from jax.experimental import pallas as pl
from jax.experimental.pallas import tpu as pltpu
from jax.experimental.pallas import tpu_sc as plsc

N = 10000
E = 160000
F = 128
B = 8
EPS = 1e-6

NC = 2
NS = 16
NW = NC * NS

E_PAD = 163840
EH_A = 65536
EH_B = E_PAD - EH_A
CHUNK = 128
SUP = 256
N_ACC = 10112
ROWS_PER_TILE = N_ACC // NS

RSQRT_F = 1.0 / math.sqrt(float(F))
DEG_NORM = 1.0 / math.sqrt(float(E) / float(N))
SQRT3 = math.sqrt(3.0)
SQRT15 = math.sqrt(15.0)
SQRT5_2 = math.sqrt(5.0) / 2.0
INV2SIG2 = 1.0 / (2.0 * 0.35 ** 2)


def _h_body(x_ref, w_ref, o_ref):
    o_ref[...] = jnp.dot(x_ref[...], w_ref[...],
                         preferred_element_type=jnp.float32) * RSQRT_F


def _input_linear(x0, W_in0):
    blk = 1000
    return pl.pallas_call(
        _h_body,
        grid=(N // blk,),
        in_specs=[pl.BlockSpec((blk, F), lambda i: (i, 0)),
                  pl.BlockSpec((F, F), lambda i: (0, 0))],
        out_specs=pl.BlockSpec((blk, F), lambda i: (i, 0)),
        out_shape=jax.ShapeDtypeStruct((N, F), jnp.float32),
    )(x0, W_in0)


def _make_gather_body(off, eh):
    nsup = eh // NW // SUP
    def body(h_hbm, idx2_hbm, out_hbm,
             idxg, ra, rb, rc,
             is0, gs0, gs1, gs2, ws0, ws1, ws2):
        c = lax.axis_index("c")
        s = lax.axis_index("s")
        wid = s * NC + c
        ebase = wid * (nsup * SUP)
        rbase = pl.multiple_of(off // CHUNK + wid * 2 * nsup, 8)
        rows = [ra, rb, rc]
        gsem = [gs0, gs1, gs2]
        wsem = [ws0, ws1, ws2]

        pltpu.sync_copy(idx2_hbm.at[pl.ds(rbase, 2 * nsup)], idxg)

        for j in range(nsup):
            b = j % 3
            if j >= 3:
                pltpu.make_async_copy(rows[b], out_hbm.at[pl.ds(0, SUP)],
                                      wsem[b]).wait()
            for q in range(2):
                pltpu.async_copy(h_hbm.at[idxg.at[2 * j + q]],
                                 rows[b].at[pl.ds(q * CHUNK, CHUNK)], gsem[b])
            for q in range(2):
                pltpu.make_async_copy(h_hbm.at[idxg.at[2 * j + q]],
                                      rows[b].at[pl.ds(q * CHUNK, CHUNK)],
                                      gsem[b]).wait()
            pltpu.async_copy(rows[b],
                             out_hbm.at[pl.ds(ebase + j * SUP, SUP)], wsem[b])

        for j in range(nsup - 3, nsup):
            b = j % 3
            pltpu.make_async_copy(rows[b], out_hbm.at[pl.ds(0, SUP)],
                                  wsem[b]).wait()
    return body


def _gather(h, src3, off, eh, half):
    mesh = plsc.VectorSubcoreMesh(core_axis_name="c", subcore_axis_name="s",
                                  num_cores=NC, num_subcores=NS)
    fn = functools.partial(
        pl.kernel,
        out_type=jax.ShapeDtypeStruct((eh, F), jnp.float32),
        mesh=mesh,
        scratch_types=(
            [pltpu.VMEM((2 * (eh // NW // SUP), CHUNK), jnp.int32)]
            + [pltpu.VMEM((SUP, F), jnp.float32)] * 3
            + [pltpu.SemaphoreType.DMA] * 7
        ),
        name=f"edge_gather_h{half}",
    )(_make_gather_body(off, eh))
    return fn(h, src3)


def _msg_body(hs_ref, ea_ref, w0_ref, w1_ref, w2_ref, o_ref):
    hs = hs_ref[...]
    ea = ea_ref[...]
    ex, ey, ez = ea[:, 0:1], ea[:, 1:2], ea[:, 2:3]
    d = jnp.sqrt(ex * ex + ey * ey + ez * ez + EPS)
    inv_d = 1.0 / d
    ux, uy, uz = ex * inv_d, ey * inv_d, ez * inv_d

    parts = []
    for b in range(B):
        cb = 2.5 * b / (B - 1)
        basis_b = jnp.exp(-((d - cb) ** 2) * INV2SIG2)
        parts.append(basis_b * hs)
    hb = jnp.concatenate(parts, axis=1).astype(jnp.bfloat16)

    s0 = jnp.dot(hb, w0_ref[...], preferred_element_type=jnp.float32) * RSQRT_F
    s1 = jnp.dot(hb, w1_ref[...], preferred_element_type=jnp.float32) * RSQRT_F
    s2 = jnp.dot(hb, w2_ref[...], preferred_element_type=jnp.float32) * RSQRT_F

    y1 = (SQRT3 * ux, SQRT3 * uy, SQRT3 * uz)
    y2 = (SQRT15 * ux * uy,
          SQRT15 * uy * uz,
          SQRT5_2 * (3.0 * uz * uz - 1.0),
          SQRT15 * ux * uz,
          (SQRT15 / 2.0) * (ux * ux - uy * uy))

    o_ref[0] = s0
    for m in range(3):
        o_ref[1 + m] = s1 * y1[m]
    for m in range(5):
        o_ref[4 + m] = s2 * y2[m]


def _edge_messages(hs, ea_full, off, Wr0f, Wr1f, Wr2f):
    blk = 512
    eh = hs.shape[0]
    ob = off // blk
    return pl.pallas_call(
        _msg_body,
        grid=(eh // blk,),
        in_specs=[pl.BlockSpec((blk, F), lambda i: (i, 0)),
                  pl.BlockSpec((blk, 3), lambda i: (i, 0)),
                  pl.BlockSpec((B * F, F), lambda i: (0, 0)),
                  pl.BlockSpec((B * F, F), lambda i: (0, 0)),
                  pl.BlockSpec((B * F, F), lambda i: (0, 0))],
        out_specs=pl.BlockSpec((9, blk, F), lambda i: (0, i, 0)),
        out_shape=jax.ShapeDtypeStruct((9, eh, F), jnp.float32),
    )(hs, ea_full, Wr0f, Wr1f, Wr2f)


def _make_scatter_body(off, eh, half):
    nch = eh // NC // NS // CHUNK
    def body(msg_hbm, dst3_hbm, init_hbm, out_hbm,
             idxall, ma, mb, acc,
             fs0, fs1, ss0, ss1):
        c = lax.axis_index("c")
        s = lax.axis_index("s")
        msgs = [ma, mb]
        fsem = [fs0, fs1]
        ssem = [ss0, ss1]
        ebase = c * (eh // NC) + s * (nch * CHUNK)
        rbase = pl.multiple_of((off + ebase) // CHUNK, 8)

        pltpu.sync_copy(dst3_hbm.at[pl.ds(rbase, nch)], idxall)

        def pass_body(p, carry):
            if half == 0:
                pltpu.sync_copy(init_hbm,
                                acc.at[pl.ds(s * ROWS_PER_TILE,
                                             ROWS_PER_TILE)])
            else:
                pltpu.sync_copy(init_hbm.at[p, c,
                                            pl.ds(s * ROWS_PER_TILE,
                                                  ROWS_PER_TILE)],
                                acc.at[pl.ds(s * ROWS_PER_TILE,
                                             ROWS_PER_TILE)])
            plsc.subcore_barrier()

            def fetch(j, b):
                pltpu.async_copy(msg_hbm.at[p, pl.ds(ebase + j * CHUNK,
                                                     CHUNK)],
                                 msgs[b], fsem[b])

            def wait_fetch(b):
                pltpu.make_async_copy(msg_hbm.at[0, pl.ds(0, CHUNK)],
                                      msgs[b], fsem[b]).wait()

            def scat(j, b):
                pltpu.async_copy(msgs[b], acc.at[idxall.at[j]], ssem[b],
                                 add=True)

            def wait_scat(j, b):
                pltpu.make_async_copy(msgs[b], acc.at[idxall.at[j]],
                                      ssem[b]).wait()

            fetch(0, 0)
            for j in range(nch):
                b = j & 1
                wait_fetch(b)
                scat(j, b)
                if j + 1 < nch:
                    b1 = 1 - b
                    if j >= 1:
                        wait_scat(j - 1, b1)
                    fetch(j + 1, b1)
            wait_scat(nch - 2, 0)
            wait_scat(nch - 1, 1)
            plsc.subcore_barrier()

            for k in range(5):
                b = k & 1
                nr = CHUNK if k < 4 else (ROWS_PER_TILE - 4 * CHUNK)
                if k >= 2:
                    pltpu.make_async_copy(msgs[b].at[pl.ds(0, CHUNK)],
                                          out_hbm.at[p, c, pl.ds(0, CHUNK)],
                                          ssem[b]).wait()
                r0 = s * ROWS_PER_TILE + k * CHUNK
                pltpu.sync_copy(acc.at[pl.ds(r0, nr)],
                                msgs[b].at[pl.ds(0, nr)])
                pltpu.async_copy(msgs[b].at[pl.ds(0, nr)],
                                 out_hbm.at[p, c, pl.ds(r0, nr)], ssem[b])
            for k in range(3, 5):
                b = k & 1
                nr = CHUNK if k < 4 else (ROWS_PER_TILE - 4 * CHUNK)
                pltpu.make_async_copy(msgs[b].at[pl.ds(0, nr)],
                                      out_hbm.at[p, c, pl.ds(0, nr)],
                                      ssem[b]).wait()
            plsc.subcore_barrier()
            return carry

        lax.fori_loop(0, 9, pass_body, 0)
    return body


def _scatter(msg, dst3, init_arr, off, eh, half):
    mesh = plsc.VectorSubcoreMesh(core_axis_name="c", subcore_axis_name="s",
                                  num_cores=NC, num_subcores=NS)
    fn = functools.partial(
        pl.kernel,
        out_type=jax.ShapeDtypeStruct((9, NC, N_ACC, F), jnp.float32),
        mesh=mesh,
        scratch_types=(
            [pltpu.VMEM((eh // NC // NS // CHUNK, CHUNK), jnp.int32)]
            + [pltpu.VMEM((CHUNK, F), jnp.float32)] * 2
            + [pltpu.VMEM_SHARED((N_ACC, F), jnp.float32)]
            + [pltpu.SemaphoreType.DMA] * 4
        ),
        name=f"edge_scatter_h{half}",
    )(_make_scatter_body(off, eh, half))
    return fn(msg, dst3, init_arr)


def _out_body(pa_ref, w0_ref, w1_ref, w2_ref, o0_ref, o1_ref, o2_ref):
    pa = pa_ref[...]
    g = (pa[:, 0] + pa[:, 1]) * DEG_NORM

    a0 = g[0]
    a1 = [g[1 + m] for m in range(3)]
    a2 = [g[4 + m] for m in range(5)]

    rms0 = jnp.sqrt(jnp.mean(a0 * a0, axis=-1, keepdims=True) + EPS)
    n0 = a0 / rms0
    ss1 = sum(jnp.sum(t * t, axis=-1, keepdims=True) for t in a1)
    rms1 = jnp.sqrt(ss1 / (3.0 * F) + EPS)
    ss2 = sum(jnp.sum(t * t, axis=-1, keepdims=True) for t in a2)
    rms2 = jnp.sqrt(ss2 / (5.0 * F) + EPS)

    o0 = jnp.dot(n0, w0_ref[...], preferred_element_type=jnp.float32) * RSQRT_F
    o0_ref[...] = jax.nn.relu(o0)

    t1 = [jnp.dot(t / rms1, w1_ref[...], preferred_element_type=jnp.float32)
          * RSQRT_F for t in a1]
    nn1 = jnp.sqrt(sum(t * t for t in t1) + EPS)
    f1 = nn1 / (nn1 + EPS)
    o1_ref[...] = jnp.concatenate([t * f1 for t in t1], axis=1)

    t2 = [jnp.dot(t / rms2, w2_ref[...], preferred_element_type=jnp.float32)
          * RSQRT_F for t in a2]
    nn2 = jnp.sqrt(sum(t * t for t in t2) + EPS)
    f2 = nn2 / (nn2 + EPS)
    o2_ref[...] = jnp.concatenate([t * f2 for t in t2], axis=1)


def _node_epilogue(part, W_out0, W_out1, W_out2):
    blk = 200
    return pl.pallas_call(
        _out_body,
        grid=(N // blk,),
        in_specs=[pl.BlockSpec((9, NC, blk, F), lambda i: (0, 0, i, 0)),
                  pl.BlockSpec((F, F), lambda i: (0, 0)),
                  pl.BlockSpec((F, F), lambda i: (0, 0)),
                  pl.BlockSpec((F, F), lambda i: (0, 0))],
        out_specs=[pl.BlockSpec((blk, F), lambda i: (i, 0)),
                   pl.BlockSpec((blk, 3 * F), lambda i: (i, 0)),
                   pl.BlockSpec((blk, 5 * F), lambda i: (i, 0))],
        out_shape=[jax.ShapeDtypeStruct((N, F), jnp.float32),
                   jax.ShapeDtypeStruct((N, 3 * F), jnp.float32),
                   jax.ShapeDtypeStruct((N, 5 * F), jnp.float32)],
    )(part, W_out0, W_out1, W_out2)


def kernel(x, edge_index, edge_attr, W_in0, W_r0, W_r1, W_r2,
           W_out0, W_out1, W_out2):
    x0 = x[0]
    src = edge_index[0]
    dst = edge_index[1]
    pad = E_PAD - E
    src_p = jnp.concatenate([src, jnp.zeros((pad,), jnp.int32)])
    dst_p = jnp.concatenate([dst, jnp.full((pad,), N, jnp.int32)])
    ea_p = jnp.concatenate([edge_attr, jnp.zeros((pad, 3), jnp.float32)])
    src2 = src_p.reshape(E_PAD // CHUNK, CHUNK)
    dst2 = dst_p.reshape(E_PAD // CHUNK, CHUNK)
    zeros = jnp.zeros((ROWS_PER_TILE, F), jnp.float32)

    Wr0f = W_r0.transpose(0, 2, 1).reshape(B * F, F).astype(jnp.bfloat16)
    Wr1f = W_r1.transpose(0, 2, 1).reshape(B * F, F).astype(jnp.bfloat16)
    Wr2f = W_r2.transpose(0, 2, 1).reshape(B * F, F).astype(jnp.bfloat16)

    h = _input_linear(x0, W_in0)
    hs_a = _gather(h, src2, 0, EH_A, 0)
    msg_a = _edge_messages(hs_a, ea_p[:EH_A], 0, Wr0f, Wr1f, Wr2f)
    hs_b = _gather(h, src2, EH_A, EH_B, 1)
    msg_b = _edge_messages(hs_b, ea_p[EH_A:], EH_A, Wr0f, Wr1f, Wr2f)
    part_a = _scatter(msg_a, dst2, zeros, 0, EH_A, 0)
    part_b = _scatter(msg_b, dst2, part_a, EH_A, EH_B, 1)
    o0, o1, o2 = _node_epilogue(part_b, W_out0, W_out1, W_out2)
    out1 = o1.reshape(N, 3, F).transpose(0, 2, 1).reshape(N, 3 * F)
    out2 = o2.reshape(N, 5, F).transpose(0, 2, 1).reshape(N, 5 * F)
    return (o0, out1, out2)
